# baseline (device time: 49075 ns/iter reference)
import jax
import jax.numpy as jnp
from jax import lax
from jax.experimental import pallas as pl
from jax.experimental.pallas import tpu as pltpu

N_DEV = 4
N_TOK = 512
D_IN = 256
D_OUT = 512
N_EXP = 16
EXP_PER_DEV = N_EXP // N_DEV
CAP = 25


def kernel(x, router_W, route_idx, expert_W):
    def body(x_ref, rw_ref, idx_ref, ew_ref, out_ref, comm_ref, send_sems, recv_sems):
        my = lax.axis_index("i")
        left = lax.rem(my + N_DEV - 1, N_DEV)
        right = lax.rem(my + 1, N_DEV)

        barrier_sem = pltpu.get_barrier_semaphore()
        for nbr in (left, right):
            pl.semaphore_signal(
                barrier_sem, inc=1,
                device_id=(nbr,), device_id_type=pl.DeviceIdType.MESH,
            )
        pl.semaphore_wait(barrier_sem, 2)

        idx = idx_ref[:, :]
        eio = lax.broadcasted_iota(jnp.int32, (N_TOK, N_EXP), 1)
        onehot = (idx == eio).astype(jnp.float32)
        row = lax.broadcasted_iota(jnp.int32, (N_TOK, N_TOK), 0)
        col = lax.broadcasted_iota(jnp.int32, (N_TOK, N_TOK), 1)
        tri = (col <= row).astype(jnp.float32)
        csum = jnp.dot(tri, onehot, preferred_element_type=jnp.float32)
        rank = jnp.sum(csum * onehot, axis=1, keepdims=True) - 1.0
        keep = rank < float(CAP)

        xv = x_ref[:, :]
        acc = jnp.zeros((N_TOK, D_OUT), jnp.float32)
        for j in range(EXP_PER_DEV):
            e = my * EXP_PER_DEV + j
            m = jnp.where((idx == e) & keep, 1.0, 0.0)
            acc = acc + jnp.dot(
                xv * m, ew_ref[j], preferred_element_type=jnp.float32
            )

        out_ref[:, :] = acc
        comm_ref[0, :, :] = acc

        for h in range(N_DEV - 1):
            s, r = h % 2, (h + 1) % 2
            rdma = pltpu.make_async_remote_copy(
                src_ref=comm_ref.at[s],
                dst_ref=comm_ref.at[r],
                send_sem=send_sems.at[s],
                recv_sem=recv_sems.at[r],
                device_id=(right,),
                device_id_type=pl.DeviceIdType.MESH,
            )
            rdma.start()
            rdma.wait()
            out_ref[:, :] = out_ref[:, :] + comm_ref[r, :, :]

    return pl.pallas_call(
        body,
        out_shape=jax.ShapeDtypeStruct((N_TOK, D_OUT), jnp.float32),
        in_specs=[pl.BlockSpec(memory_space=pltpu.VMEM)] * 4,
        out_specs=pl.BlockSpec(memory_space=pltpu.VMEM),
        scratch_shapes=[
            pltpu.VMEM((2, N_TOK, D_OUT), jnp.float32),
            pltpu.SemaphoreType.DMA((2,)),
            pltpu.SemaphoreType.DMA((2,)),
        ],
        compiler_params=pltpu.CompilerParams(collective_id=0),
    )(x, router_W, route_idx, expert_W)


# device time: 24302 ns/iter; 2.0194x vs baseline; 2.0194x over previous
import jax
import jax.numpy as jnp
from jax import lax
from jax.experimental import pallas as pl
from jax.experimental.pallas import tpu as pltpu

N_DEV = 4
N_TOK = 512
D_IN = 256
D_OUT = 512
N_EXP = 16
EXP_PER_DEV = N_EXP // N_DEV
CAP = 25


HALF = D_OUT // 2


def kernel(x, router_W, route_idx, expert_W):
    def body(
        x_ref, rw_ref, idx_ref, ew_ref, out_ref,
        bufA_ref, bufB_ref, recvA_ref, recvB_ref,
        sendA_sems, recvA_sems, sendB_sems, recvB_sems,
    ):
        my = lax.axis_index("i")
        yp = my ^ 1
        xp = 3 - my

        barrier_sem = pltpu.get_barrier_semaphore()
        for nbr in (yp, xp):
            pl.semaphore_signal(
                barrier_sem, inc=1,
                device_id=(nbr,), device_id_type=pl.DeviceIdType.MESH,
            )
        pl.semaphore_wait(barrier_sem, 2)

        idx = idx_ref[:, :]
        eio = lax.broadcasted_iota(jnp.int32, (N_TOK, N_EXP), 1)
        onehot = (idx == eio).astype(jnp.float32)
        row = lax.broadcasted_iota(jnp.int32, (N_TOK, N_TOK), 0)
        col = lax.broadcasted_iota(jnp.int32, (N_TOK, N_TOK), 1)
        tri = (col <= row).astype(jnp.float32)
        csum = jnp.dot(tri, onehot, preferred_element_type=jnp.float32)
        rank = jnp.sum(csum * onehot, axis=1, keepdims=True) - 1.0
        keep = rank < float(CAP)

        xv = x_ref[:, :]
        xm = []
        for j in range(EXP_PER_DEV):
            e = my * EXP_PER_DEV + j
            m = jnp.where((idx == e) & keep, 1.0, 0.0)
            xm.append(xv * m)
        accA = jnp.zeros((N_TOK, HALF), jnp.float32)
        accB = jnp.zeros((N_TOK, HALF), jnp.float32)
        for j in range(EXP_PER_DEV):
            accA = accA + jnp.dot(
                xm[j], ew_ref[j, :, :HALF], preferred_element_type=jnp.float32
            )
            accB = accB + jnp.dot(
                xm[j], ew_ref[j, :, HALF:], preferred_element_type=jnp.float32
            )
        bufA_ref[:, :] = accA
        bufB_ref[:, :] = accB

        def exchange(stage, a_partner, b_partner):
            rdmaA = pltpu.make_async_remote_copy(
                src_ref=bufA_ref,
                dst_ref=recvA_ref.at[stage],
                send_sem=sendA_sems.at[stage],
                recv_sem=recvA_sems.at[stage],
                device_id=(a_partner,),
                device_id_type=pl.DeviceIdType.MESH,
            )
            rdmaB = pltpu.make_async_remote_copy(
                src_ref=bufB_ref,
                dst_ref=recvB_ref.at[stage],
                send_sem=sendB_sems.at[stage],
                recv_sem=recvB_sems.at[stage],
                device_id=(b_partner,),
                device_id_type=pl.DeviceIdType.MESH,
            )
            rdmaA.start()
            rdmaB.start()
            rdmaA.wait()
            rdmaB.wait()

        exchange(0, yp, xp)
        bufA_ref[:, :] = bufA_ref[:, :] + recvA_ref[0, :, :]
        bufB_ref[:, :] = bufB_ref[:, :] + recvB_ref[0, :, :]

        exchange(1, xp, yp)
        out_ref[:, :HALF] = bufA_ref[:, :] + recvA_ref[1, :, :]
        out_ref[:, HALF:] = bufB_ref[:, :] + recvB_ref[1, :, :]

    return pl.pallas_call(
        body,
        out_shape=jax.ShapeDtypeStruct((N_TOK, D_OUT), jnp.float32),
        in_specs=[pl.BlockSpec(memory_space=pltpu.VMEM)] * 4,
        out_specs=pl.BlockSpec(memory_space=pltpu.VMEM),
        scratch_shapes=[
            pltpu.VMEM((N_TOK, HALF), jnp.float32),
            pltpu.VMEM((N_TOK, HALF), jnp.float32),
            pltpu.VMEM((2, N_TOK, HALF), jnp.float32),
            pltpu.VMEM((2, N_TOK, HALF), jnp.float32),
            pltpu.SemaphoreType.DMA((2,)),
            pltpu.SemaphoreType.DMA((2,)),
            pltpu.SemaphoreType.DMA((2,)),
            pltpu.SemaphoreType.DMA((2,)),
        ],
        compiler_params=pltpu.CompilerParams(collective_id=0),
    )(x, router_W, route_idx, expert_W)
